# Initial kernel scaffold; baseline (speedup 1.0000x reference)
import functools

import jax
import jax.numpy as jnp
from jax import lax
from jax.experimental import pallas as pl
from jax.experimental.pallas import tpu as pltpu
from jax.experimental.pallas import tpu_sc as plsc

_NC, _NS, _L = 2, 16, 16
_NW = _NC * _NS

_N = 10000
_E = 320000

_CW = 80
_ER = _E // _CW
_RPW = _ER // _NW
_RPC = 5
_NCHUNK = _RPW // _RPC
_CE = _RPC * _CW
_RPT = _N // _NS

_mesh = plsc.VectorSubcoreMesh(core_axis_name="c", subcore_axis_name="s")


def _deg_body(dst_hbm, out_hbm, dst_v, hist_v, acc_sh):
    c = lax.axis_index("c")
    s = lax.axis_index("s")
    w = c * _NS + s

    def zero_body(i, carry):
        hist_v[pl.ds(i * _L, _L)] = jnp.zeros((_L,), jnp.float32)
        return carry

    lax.fori_loop(0, _N // _L, zero_body, 0)

    pltpu.sync_copy(dst_hbm.at[pl.ds(w * _RPW, _RPW)], dst_v)

    ones = jnp.ones((_L,), jnp.float32)

    def hist_body(r, carry):
        for k in range(_CW // _L):
            idx = dst_v[r, pl.ds(k * _L, _L)]
            plsc.addupdate_scatter(hist_v, [idx], ones)
        return carry

    lax.fori_loop(0, _RPW, hist_body, 0)

    @pl.when(s == 0)
    def _():
        pltpu.sync_copy(hist_v, acc_sh)

    plsc.subcore_barrier()

    @pl.when(s != 0)
    def _():
        pltpu.sync_copy(hist_v, acc_sh, add=True)

    plsc.subcore_barrier()

    @pl.when(s == 0)
    def _():
        pltpu.sync_copy(acc_sh, out_hbm.at[pl.ds(c * _N, _N)])


_deg_call = pl.kernel(
    _deg_body,
    out_type=jax.ShapeDtypeStruct((_NC * _N,), jnp.float32),
    mesh=_mesh,
    scratch_types=[
        pltpu.VMEM((_RPW, _CW), jnp.int32),
        pltpu.VMEM((_N,), jnp.float32),
        pltpu.VMEM_SHARED((_N,), jnp.float32),
    ],
)


def _agg_body(g_hbm, src_hbm, dst_hbm, out_hbm, idxs_v, idxd_v, rows_v, acc_sh,
              sem, *, D):
    c = lax.axis_index("c")
    s = lax.axis_index("s")
    w = c * _NS + s

    def zero_body(r, carry):
        for k in range(D // _L):
            rows_v[r, pl.ds(k * _L, _L)] = jnp.zeros((_L,), jnp.float32)
        return carry

    lax.fori_loop(0, _CE, zero_body, 0)

    base0 = s * _RPT
    pltpu.sync_copy(rows_v.at[pl.ds(0, _CE)], acc_sh.at[pl.ds(base0, _CE)])
    pltpu.sync_copy(rows_v.at[pl.ds(0, _RPT - _CE)],
                    acc_sh.at[pl.ds(base0 + _CE, _RPT - _CE)])
    plsc.subcore_barrier()

    def chunk_body(j, carry):
        rbase = w * _RPW + j * _RPC
        pltpu.sync_copy(src_hbm.at[pl.ds(rbase, _RPC)], idxs_v)
        pltpu.sync_copy(dst_hbm.at[pl.ds(rbase, _RPC)], idxd_v)
        descs = []
        for k in range(_RPC):
            descs.append(
                pltpu.async_copy(g_hbm.at[idxs_v.at[k]],
                                 rows_v.at[pl.ds(k * _CW, _CW)], sem))
        for d in descs:
            d.wait()
        for k in range(_RPC):
            pltpu.sync_copy(rows_v.at[pl.ds(k * _CW, _CW)],
                            acc_sh.at[idxd_v.at[k]], add=True)
        return carry

    lax.fori_loop(0, _NCHUNK, chunk_body, 0)
    plsc.subcore_barrier()

    pltpu.sync_copy(acc_sh.at[pl.ds(s * _RPT, _RPT)],
                    out_hbm.at[pl.ds(c * _N + s * _RPT, _RPT)])


def _make_agg(D):
    return pl.kernel(
        functools.partial(_agg_body, D=D),
        out_type=jax.ShapeDtypeStruct((_NC * _N, D), jnp.float32),
        mesh=_mesh,
        scratch_types=[
            pltpu.VMEM((_RPC, _CW), jnp.int32),
            pltpu.VMEM((_RPC, _CW), jnp.int32),
            pltpu.VMEM((_CE, D), jnp.float32),
            pltpu.VMEM_SHARED((_N, D), jnp.float32),
            pltpu.SemaphoreType.DMA,
        ],
    )


_agg64 = _make_agg(64)
_agg32 = _make_agg(32)


_RB = 1000
_GRID = (_N // _RB,)


def _k1_body(x_ref, w_ref, d0_ref, d1_ref, g_ref, dinv_ref):
    deg = d0_ref[...] + d1_ref[...] + 1.0
    dv = lax.rsqrt(deg)
    h = jnp.dot(x_ref[...], w_ref[...], preferred_element_type=jnp.float32)
    g_ref[...] = h * dv
    dinv_ref[...] = dv


def _k1(x, W1, d0, d1):
    F = x.shape[1]
    Do = W1.shape[1]
    return pl.pallas_call(
        _k1_body,
        grid=_GRID,
        in_specs=[
            pl.BlockSpec((_RB, F), lambda i: (i, 0)),
            pl.BlockSpec((F, Do), lambda i: (0, 0)),
            pl.BlockSpec((_RB, 1), lambda i: (i, 0)),
            pl.BlockSpec((_RB, 1), lambda i: (i, 0)),
        ],
        out_specs=[
            pl.BlockSpec((_RB, Do), lambda i: (i, 0)),
            pl.BlockSpec((_RB, 1), lambda i: (i, 0)),
        ],
        out_shape=[
            jax.ShapeDtypeStruct((_N, Do), jnp.float32),
            jax.ShapeDtypeStruct((_N, 1), jnp.float32),
        ],
    )(x, W1, d0, d1)


def _k2_body(p0_ref, p1_ref, g_ref, dinv_ref, b_ref, w_ref, out_ref):
    sacc = p0_ref[...] + p1_ref[...] + g_ref[...]
    dv = dinv_ref[...]
    h = jnp.maximum(sacc * dv + b_ref[...], 0.0)
    out_ref[...] = jnp.dot(h, w_ref[...],
                           preferred_element_type=jnp.float32) * dv


def _k2(p0, p1, g1, dinv, b1, W2):
    Di = p0.shape[1]
    Do = W2.shape[1]
    return pl.pallas_call(
        _k2_body,
        grid=_GRID,
        in_specs=[
            pl.BlockSpec((_RB, Di), lambda i: (i, 0)),
            pl.BlockSpec((_RB, Di), lambda i: (i, 0)),
            pl.BlockSpec((_RB, Di), lambda i: (i, 0)),
            pl.BlockSpec((_RB, 1), lambda i: (i, 0)),
            pl.BlockSpec((1, Di), lambda i: (0, 0)),
            pl.BlockSpec((Di, Do), lambda i: (0, 0)),
        ],
        out_specs=pl.BlockSpec((_RB, Do), lambda i: (i, 0)),
        out_shape=jax.ShapeDtypeStruct((_N, Do), jnp.float32),
    )(p0, p1, g1, dinv, b1, W2)


def _k3_body(q0_ref, q1_ref, g_ref, dinv_ref, b_ref, w_ref, bc_ref, out_ref):
    sacc = q0_ref[...] + q1_ref[...] + g_ref[...]
    h = jnp.maximum(sacc * dinv_ref[...] + b_ref[...], 0.0)
    out_ref[...] = jnp.dot(h, w_ref[...],
                           preferred_element_type=jnp.float32) + bc_ref[...]


def _k3(q0, q1, g2, dinv, b2, Wc, bc):
    Di = q0.shape[1]
    Do = Wc.shape[1]
    return pl.pallas_call(
        _k3_body,
        grid=_GRID,
        in_specs=[
            pl.BlockSpec((_RB, Di), lambda i: (i, 0)),
            pl.BlockSpec((_RB, Di), lambda i: (i, 0)),
            pl.BlockSpec((_RB, Di), lambda i: (i, 0)),
            pl.BlockSpec((_RB, 1), lambda i: (i, 0)),
            pl.BlockSpec((1, Di), lambda i: (0, 0)),
            pl.BlockSpec((Di, Do), lambda i: (0, 0)),
            pl.BlockSpec((1, Do), lambda i: (0, 0)),
        ],
        out_specs=pl.BlockSpec((_RB, Do), lambda i: (i, 0)),
        out_shape=jax.ShapeDtypeStruct((_N, Do), jnp.float32),
    )(q0, q1, g2, dinv, b2, Wc, bc)


@jax.jit
def kernel(x, edge_index, W1, b1, W2, b2, Wc, bc):
    src = edge_index[0].reshape(_ER, _CW)
    dst = edge_index[1].reshape(_ER, _CW)

    degp = _deg_call(dst)
    d0 = degp[:_N, None]
    d1 = degp[_N:, None]

    g1, dinv = _k1(x, W1, d0, d1)
    p = _agg64(g1, src, dst)
    g2 = _k2(p[:_N], p[_N:], g1, dinv, b1.reshape(1, -1), W2)
    q = _agg32(g2, src, dst)
    return _k3(q[:_N], q[_N:], g2, dinv, b2.reshape(1, -1), Wc,
               bc.reshape(1, -1))


# speedup vs baseline: 20.5449x; 20.5449x over previous
import functools

import jax
import jax.numpy as jnp
from jax import lax
from jax.experimental import pallas as pl
from jax.experimental.pallas import tpu as pltpu
from jax.experimental.pallas import tpu_sc as plsc

_NC, _NS, _L = 2, 16, 16
_NW = _NC * _NS

_N = 10000
_E = 320000

_CW = 128
_EPW = _E // _NW
_EPWP = 10240
_PAD = _EPWP - _EPW
_RPW = _EPWP // _CW
_ER = _NW * _RPW
_RPC = 8
_NCHUNK = _RPW // _RPC
_CE = _RPC * _CW

_NP = 10240
_RPT = _NP // _NS
_CP = _NP // _NS

_mesh = plsc.VectorSubcoreMesh(core_axis_name="c", subcore_axis_name="s")


def _deg_body(dst_hbm, out_hbm, dst_v, hist_v, tmp_v, acc_v, slots_sh):
    c = lax.axis_index("c")
    s = lax.axis_index("s")
    w = c * _NS + s

    def zero_body(i, carry):
        hist_v[pl.ds(i * _L, _L)] = jnp.zeros((_L,), jnp.float32)
        return carry

    lax.fori_loop(0, _NP // _L, zero_body, 0)

    pltpu.sync_copy(dst_hbm.at[pl.ds(w * _RPW, _RPW)], dst_v)

    ones = jnp.ones((_L,), jnp.float32)

    def hist_body(r, carry):
        for k in range(_CW // _L):
            idx = dst_v[r, pl.ds(k * _L, _L)]
            plsc.addupdate_scatter(hist_v, [idx], ones)
        return carry

    lax.fori_loop(0, _RPW, hist_body, 0)

    pltpu.sync_copy(hist_v, slots_sh.at[s])
    plsc.subcore_barrier()

    for k in range(_CP // _L):
        acc_v[pl.ds(k * _L, _L)] = jnp.zeros((_L,), jnp.float32)
    for t in range(_NS):
        pltpu.sync_copy(slots_sh.at[t, pl.ds(s * _CP, _CP)], tmp_v)
        for k in range(_CP // _L):
            sl = pl.ds(k * _L, _L)
            acc_v[sl] = acc_v[sl] + tmp_v[sl]

    pltpu.sync_copy(acc_v, out_hbm.at[pl.ds(c * _NP + s * _CP, _CP)])


_deg_call = pl.kernel(
    _deg_body,
    out_type=jax.ShapeDtypeStruct((_NC * _NP,), jnp.float32),
    mesh=_mesh,
    compiler_params=pltpu.CompilerParams(needs_layout_passes=False),
    scratch_types=[
        pltpu.VMEM((_RPW, _CW), jnp.int32),
        pltpu.VMEM((_NP,), jnp.float32),
        pltpu.VMEM((_CP,), jnp.float32),
        pltpu.VMEM((_CP,), jnp.float32),
        pltpu.VMEM_SHARED((_NS, _NP), jnp.float32),
    ],
)


def _agg_body(g_hbm, src_hbm, dst_hbm, out_hbm, idxs_v, idxd_v, rows_v, acc_sh,
              sem, *, D):
    c = lax.axis_index("c")
    s = lax.axis_index("s")
    w = c * _NS + s

    def zero_body(r, carry):
        for k in range(D // _L):
            rows_v[r, pl.ds(k * _L, _L)] = jnp.zeros((_L,), jnp.float32)
        return carry

    lax.fori_loop(0, _CE, zero_body, 0)

    pltpu.sync_copy(rows_v.at[pl.ds(0, _RPT)], acc_sh.at[pl.ds(s * _RPT, _RPT)])
    plsc.subcore_barrier()

    def chunk_body(j, carry):
        rbase = w * _RPW + j * _RPC
        pltpu.sync_copy(src_hbm.at[pl.ds(rbase, _RPC)], idxs_v)
        pltpu.sync_copy(dst_hbm.at[pl.ds(rbase, _RPC)], idxd_v)
        descs = []
        for k in range(_RPC):
            descs.append(
                pltpu.async_copy(g_hbm.at[idxs_v.at[k]],
                                 rows_v.at[pl.ds(k * _CW, _CW)], sem))
        for k in range(_RPC):
            descs[k].wait()
            pltpu.sync_copy(rows_v.at[pl.ds(k * _CW, _CW)],
                            acc_sh.at[idxd_v.at[k]], add=True)
        return carry

    lax.fori_loop(0, _NCHUNK, chunk_body, 0)
    plsc.subcore_barrier()

    pltpu.sync_copy(acc_sh.at[pl.ds(s * _RPT, _RPT)],
                    out_hbm.at[pl.ds(c * _NP + s * _RPT, _RPT)])


def _make_agg(D):
    return pl.kernel(
        functools.partial(_agg_body, D=D),
        out_type=jax.ShapeDtypeStruct((_NC * _NP, D), jnp.float32),
        mesh=_mesh,
        compiler_params=pltpu.CompilerParams(use_tc_tiling_on_sc=False),
        scratch_types=[
            pltpu.VMEM((_RPC, _CW), jnp.int32),
            pltpu.VMEM((_RPC, _CW), jnp.int32),
            pltpu.VMEM((_CE, D), jnp.float32),
            pltpu.VMEM_SHARED((_NP, D), jnp.float32),
            pltpu.SemaphoreType.DMA,
        ],
    )


_agg64 = _make_agg(64)
_agg32 = _make_agg(32)


_RB = 1000
_GRID = (_N // _RB,)


def _k1_body(x_ref, w_ref, d0_ref, d1_ref, g_ref, dinv_ref):
    deg = d0_ref[...] + d1_ref[...] + 1.0
    dv = lax.rsqrt(deg)
    h = jnp.dot(x_ref[...], w_ref[...], preferred_element_type=jnp.float32)
    g_ref[...] = h * dv
    dinv_ref[...] = dv


def _k1(x, W1, d0, d1):
    F = x.shape[1]
    Do = W1.shape[1]
    return pl.pallas_call(
        _k1_body,
        grid=_GRID,
        in_specs=[
            pl.BlockSpec((_RB, F), lambda i: (i, 0)),
            pl.BlockSpec((F, Do), lambda i: (0, 0)),
            pl.BlockSpec((_RB, 1), lambda i: (i, 0)),
            pl.BlockSpec((_RB, 1), lambda i: (i, 0)),
        ],
        out_specs=[
            pl.BlockSpec((_RB, Do), lambda i: (i, 0)),
            pl.BlockSpec((_RB, 1), lambda i: (i, 0)),
        ],
        out_shape=[
            jax.ShapeDtypeStruct((_N, Do), jnp.float32),
            jax.ShapeDtypeStruct((_N, 1), jnp.float32),
        ],
    )(x, W1, d0, d1)


def _k2_body(p0_ref, p1_ref, g_ref, dinv_ref, b_ref, w_ref, out_ref):
    sacc = p0_ref[...] + p1_ref[...] + g_ref[...]
    dv = dinv_ref[...]
    h = jnp.maximum(sacc * dv + b_ref[...], 0.0)
    out_ref[...] = jnp.dot(h, w_ref[...],
                           preferred_element_type=jnp.float32) * dv


def _k2(p0, p1, g1, dinv, b1, W2):
    Di = p0.shape[1]
    Do = W2.shape[1]
    return pl.pallas_call(
        _k2_body,
        grid=_GRID,
        in_specs=[
            pl.BlockSpec((_RB, Di), lambda i: (i, 0)),
            pl.BlockSpec((_RB, Di), lambda i: (i, 0)),
            pl.BlockSpec((_RB, Di), lambda i: (i, 0)),
            pl.BlockSpec((_RB, 1), lambda i: (i, 0)),
            pl.BlockSpec((1, Di), lambda i: (0, 0)),
            pl.BlockSpec((Di, Do), lambda i: (0, 0)),
        ],
        out_specs=pl.BlockSpec((_RB, Do), lambda i: (i, 0)),
        out_shape=jax.ShapeDtypeStruct((_N, Do), jnp.float32),
    )(p0, p1, g1, dinv, b1, W2)


def _k3_body(q0_ref, q1_ref, g_ref, dinv_ref, b_ref, w_ref, bc_ref, out_ref):
    sacc = q0_ref[...] + q1_ref[...] + g_ref[...]
    h = jnp.maximum(sacc * dinv_ref[...] + b_ref[...], 0.0)
    out_ref[...] = jnp.dot(h, w_ref[...],
                           preferred_element_type=jnp.float32) + bc_ref[...]


def _k3(q0, q1, g2, dinv, b2, Wc, bc):
    Di = q0.shape[1]
    Do = Wc.shape[1]
    return pl.pallas_call(
        _k3_body,
        grid=_GRID,
        in_specs=[
            pl.BlockSpec((_RB, Di), lambda i: (i, 0)),
            pl.BlockSpec((_RB, Di), lambda i: (i, 0)),
            pl.BlockSpec((_RB, Di), lambda i: (i, 0)),
            pl.BlockSpec((_RB, 1), lambda i: (i, 0)),
            pl.BlockSpec((1, Di), lambda i: (0, 0)),
            pl.BlockSpec((Di, Do), lambda i: (0, 0)),
            pl.BlockSpec((1, Do), lambda i: (0, 0)),
        ],
        out_specs=pl.BlockSpec((_RB, Do), lambda i: (i, 0)),
        out_shape=jax.ShapeDtypeStruct((_N, Do), jnp.float32),
    )(q0, q1, g2, dinv, b2, Wc, bc)


@jax.jit
def kernel(x, edge_index, W1, b1, W2, b2, Wc, bc):
    src = jnp.pad(edge_index[0].reshape(_NW, _EPW), ((0, 0), (0, _PAD)),
                  constant_values=0).reshape(_ER, _CW)
    dst = jnp.pad(edge_index[1].reshape(_NW, _EPW), ((0, 0), (0, _PAD)),
                  constant_values=_NP - 1).reshape(_ER, _CW)

    degp = _deg_call(dst)
    d0 = degp[:_N, None]
    d1 = degp[_NP:_NP + _N, None]

    g1, dinv = _k1(x, W1, d0, d1)
    p = _agg64(g1, src, dst)
    g2 = _k2(p[:_N], p[_NP:_NP + _N], g1, dinv, b1.reshape(1, -1), W2)
    q = _agg32(g2, src, dst)
    return _k3(q[:_N], q[_NP:_NP + _N], g2, dinv, b2.reshape(1, -1), Wc,
               bc.reshape(1, -1))


# speedup vs baseline: 21.8086x; 1.0615x over previous
import functools

import jax
import jax.numpy as jnp
from jax import lax
from jax.experimental import pallas as pl
from jax.experimental.pallas import tpu as pltpu
from jax.experimental.pallas import tpu_sc as plsc

_NC, _NS, _L = 2, 16, 16
_NW = _NC * _NS

_N = 10000
_E = 320000

_CW = 128
_EPW = _E // _NW
_EPWP = 10240
_PAD = _EPWP - _EPW
_RPW = _EPWP // _CW
_ER = _NW * _RPW
_RPC = 8
_NCHUNK = _RPW // _RPC
_CE = _RPC * _CW

_NP = 10240
_RPT = _NP // _NS
_CP = _NP // _NS

_mesh = plsc.VectorSubcoreMesh(core_axis_name="c", subcore_axis_name="s")


def _deg_body(dst_hbm, out_hbm, dst_v, hist_v, tmp_v, acc_v, slots_sh):
    c = lax.axis_index("c")
    s = lax.axis_index("s")
    w = c * _NS + s

    def zero_body(i, carry):
        hist_v[pl.ds(i * _L, _L)] = jnp.zeros((_L,), jnp.float32)
        return carry

    lax.fori_loop(0, _NP // _L, zero_body, 0)

    pltpu.sync_copy(dst_hbm.at[pl.ds(w * _RPW, _RPW)], dst_v)

    ones = jnp.ones((_L,), jnp.float32)

    def hist_body(r, carry):
        for k in range(_CW // _L):
            idx = dst_v[r, pl.ds(k * _L, _L)]
            plsc.addupdate_scatter(hist_v, [idx], ones)
        return carry

    lax.fori_loop(0, _RPW, hist_body, 0)

    pltpu.sync_copy(hist_v, slots_sh.at[s])
    plsc.subcore_barrier()

    for k in range(_CP // _L):
        acc_v[pl.ds(k * _L, _L)] = jnp.zeros((_L,), jnp.float32)
    for t in range(_NS):
        pltpu.sync_copy(slots_sh.at[t, pl.ds(s * _CP, _CP)], tmp_v)
        for k in range(_CP // _L):
            sl = pl.ds(k * _L, _L)
            acc_v[sl] = acc_v[sl] + tmp_v[sl]

    pltpu.sync_copy(acc_v, out_hbm.at[pl.ds(c * _NP + s * _CP, _CP)])


_deg_call = pl.kernel(
    _deg_body,
    out_type=jax.ShapeDtypeStruct((_NC * _NP,), jnp.float32),
    mesh=_mesh,
    compiler_params=pltpu.CompilerParams(needs_layout_passes=False),
    scratch_types=[
        pltpu.VMEM((_RPW, _CW), jnp.int32),
        pltpu.VMEM((_NP,), jnp.float32),
        pltpu.VMEM((_CP,), jnp.float32),
        pltpu.VMEM((_CP,), jnp.float32),
        pltpu.VMEM_SHARED((_NS, _NP), jnp.float32),
    ],
)


_SR = 4
_SCE = _SR * _CW
_NSUB = _RPW // _SR
_NPAIR = _NSUB // 2


def _agg_body(g_hbm, src_hbm, dst_hbm, out_hbm, idxs_v, idxd_v, rows0_v,
              rows1_v, acc_sh, g0, g1, s0, s1, *, D):
    c = lax.axis_index("c")
    s = lax.axis_index("s")
    w = c * _NS + s

    def zero_body(r, carry):
        for k in range(D // _L):
            rows0_v[r, pl.ds(k * _L, _L)] = jnp.zeros((_L,), jnp.float32)
        return carry

    lax.fori_loop(0, _SCE, zero_body, 0)
    pltpu.sync_copy(rows0_v, acc_sh.at[pl.ds(s * _RPT, _SCE)])
    pltpu.sync_copy(rows0_v.at[pl.ds(0, _RPT - _SCE)],
                    acc_sh.at[pl.ds(s * _RPT + _SCE, _RPT - _SCE)])
    plsc.subcore_barrier()

    pltpu.sync_copy(src_hbm.at[pl.ds(w * _RPW, _RPW)], idxs_v)
    pltpu.sync_copy(dst_hbm.at[pl.ds(w * _RPW, _RPW)], idxd_v)

    def fire_gather(rbase, rows_v, sem):
        for k in range(_SR):
            pltpu.async_copy(g_hbm.at[idxs_v.at[rbase + k]],
                             rows_v.at[pl.ds(k * _CW, _CW)], sem)

    def fire_scatter(rbase, rows_v, sem):
        for k in range(_SR):
            pltpu.async_copy(rows_v.at[pl.ds(k * _CW, _CW)],
                             acc_sh.at[idxd_v.at[rbase + k]], sem, add=True)

    def drain(sem, rows_v):
        pltpu.make_async_copy(g_hbm.at[pl.ds(0, _SCE)], rows_v, sem).wait()

    fire_gather(0, rows0_v, g0)

    def pair_body(i, carry):
        base = 2 * i * _SR
        @pl.when(i > 0)
        def _():
            drain(s1, rows1_v)
        fire_gather(base + _SR, rows1_v, g1)
        drain(g0, rows0_v)
        fire_scatter(base, rows0_v, s0)
        drain(s0, rows0_v)
        @pl.when(i < _NPAIR - 1)
        def _():
            fire_gather(base + 2 * _SR, rows0_v, g0)
        drain(g1, rows1_v)
        fire_scatter(base + _SR, rows1_v, s1)
        return carry

    lax.fori_loop(0, _NPAIR, pair_body, 0)
    drain(s1, rows1_v)
    plsc.subcore_barrier()

    pltpu.sync_copy(acc_sh.at[pl.ds(s * _RPT, _RPT)],
                    out_hbm.at[pl.ds(c * _NP + s * _RPT, _RPT)])


def _make_agg(D):
    return pl.kernel(
        functools.partial(_agg_body, D=D),
        out_type=jax.ShapeDtypeStruct((_NC * _NP, D), jnp.float32),
        mesh=_mesh,
        compiler_params=pltpu.CompilerParams(use_tc_tiling_on_sc=False),
        scratch_types=[
            pltpu.VMEM((_RPW, _CW), jnp.int32),
            pltpu.VMEM((_RPW, _CW), jnp.int32),
            pltpu.VMEM((_SCE, D), jnp.float32),
            pltpu.VMEM((_SCE, D), jnp.float32),
            pltpu.VMEM_SHARED((_NP, D), jnp.float32),
            pltpu.SemaphoreType.DMA,
            pltpu.SemaphoreType.DMA,
            pltpu.SemaphoreType.DMA,
            pltpu.SemaphoreType.DMA,
        ],
    )


_agg64 = _make_agg(64)
_agg32 = _make_agg(32)


_RB = 1000
_GRID = (_N // _RB,)


def _k1_body(x_ref, w_ref, d0_ref, d1_ref, g_ref, dinv_ref):
    deg = d0_ref[...] + d1_ref[...] + 1.0
    dv = lax.rsqrt(deg)
    h = jnp.dot(x_ref[...], w_ref[...], preferred_element_type=jnp.float32)
    g_ref[...] = h * dv
    dinv_ref[...] = dv


def _k1(x, W1, d0, d1):
    F = x.shape[1]
    Do = W1.shape[1]
    return pl.pallas_call(
        _k1_body,
        grid=_GRID,
        in_specs=[
            pl.BlockSpec((_RB, F), lambda i: (i, 0)),
            pl.BlockSpec((F, Do), lambda i: (0, 0)),
            pl.BlockSpec((_RB, 1), lambda i: (i, 0)),
            pl.BlockSpec((_RB, 1), lambda i: (i, 0)),
        ],
        out_specs=[
            pl.BlockSpec((_RB, Do), lambda i: (i, 0)),
            pl.BlockSpec((_RB, 1), lambda i: (i, 0)),
        ],
        out_shape=[
            jax.ShapeDtypeStruct((_N, Do), jnp.float32),
            jax.ShapeDtypeStruct((_N, 1), jnp.float32),
        ],
    )(x, W1, d0, d1)


def _k2_body(p0_ref, p1_ref, g_ref, dinv_ref, b_ref, w_ref, out_ref):
    sacc = p0_ref[...] + p1_ref[...] + g_ref[...]
    dv = dinv_ref[...]
    h = jnp.maximum(sacc * dv + b_ref[...], 0.0)
    out_ref[...] = jnp.dot(h, w_ref[...],
                           preferred_element_type=jnp.float32) * dv


def _k2(p0, p1, g1, dinv, b1, W2):
    Di = p0.shape[1]
    Do = W2.shape[1]
    return pl.pallas_call(
        _k2_body,
        grid=_GRID,
        in_specs=[
            pl.BlockSpec((_RB, Di), lambda i: (i, 0)),
            pl.BlockSpec((_RB, Di), lambda i: (i, 0)),
            pl.BlockSpec((_RB, Di), lambda i: (i, 0)),
            pl.BlockSpec((_RB, 1), lambda i: (i, 0)),
            pl.BlockSpec((1, Di), lambda i: (0, 0)),
            pl.BlockSpec((Di, Do), lambda i: (0, 0)),
        ],
        out_specs=pl.BlockSpec((_RB, Do), lambda i: (i, 0)),
        out_shape=jax.ShapeDtypeStruct((_N, Do), jnp.float32),
    )(p0, p1, g1, dinv, b1, W2)


def _k3_body(q0_ref, q1_ref, g_ref, dinv_ref, b_ref, w_ref, bc_ref, out_ref):
    sacc = q0_ref[...] + q1_ref[...] + g_ref[...]
    h = jnp.maximum(sacc * dinv_ref[...] + b_ref[...], 0.0)
    out_ref[...] = jnp.dot(h, w_ref[...],
                           preferred_element_type=jnp.float32) + bc_ref[...]


def _k3(q0, q1, g2, dinv, b2, Wc, bc):
    Di = q0.shape[1]
    Do = Wc.shape[1]
    return pl.pallas_call(
        _k3_body,
        grid=_GRID,
        in_specs=[
            pl.BlockSpec((_RB, Di), lambda i: (i, 0)),
            pl.BlockSpec((_RB, Di), lambda i: (i, 0)),
            pl.BlockSpec((_RB, Di), lambda i: (i, 0)),
            pl.BlockSpec((_RB, 1), lambda i: (i, 0)),
            pl.BlockSpec((1, Di), lambda i: (0, 0)),
            pl.BlockSpec((Di, Do), lambda i: (0, 0)),
            pl.BlockSpec((1, Do), lambda i: (0, 0)),
        ],
        out_specs=pl.BlockSpec((_RB, Do), lambda i: (i, 0)),
        out_shape=jax.ShapeDtypeStruct((_N, Do), jnp.float32),
    )(q0, q1, g2, dinv, b2, Wc, bc)


@jax.jit
def kernel(x, edge_index, W1, b1, W2, b2, Wc, bc):
    src = jnp.pad(edge_index[0].reshape(_NW, _EPW), ((0, 0), (0, _PAD)),
                  constant_values=0).reshape(_ER, _CW)
    dst = jnp.pad(edge_index[1].reshape(_NW, _EPW), ((0, 0), (0, _PAD)),
                  constant_values=_NP - 1).reshape(_ER, _CW)

    degp = _deg_call(dst)
    d0 = degp[:_N, None]
    d1 = degp[_NP:_NP + _N, None]

    g1, dinv = _k1(x, W1, d0, d1)
    p = _agg64(g1, src, dst)
    g2 = _k2(p[:_N], p[_NP:_NP + _N], g1, dinv, b1.reshape(1, -1), W2)
    q = _agg32(g2, src, dst)
    return _k3(q[:_N], q[_NP:_NP + _N], g2, dinv, b2.reshape(1, -1), Wc,
               bc.reshape(1, -1))


# speedup vs baseline: 25.2483x; 1.1577x over previous
import functools

import jax
import jax.numpy as jnp
from jax import lax
from jax.experimental import pallas as pl
from jax.experimental.pallas import tpu as pltpu
from jax.experimental.pallas import tpu_sc as plsc

_NC, _NS, _L = 2, 16, 16
_NW = _NC * _NS

_N = 10000
_E = 320000

_CW = 128
_EPW = _E // _NW
_EPWP = 10240
_PAD = _EPWP - _EPW
_RPW = _EPWP // _CW
_ER = _NW * _RPW
_RPC = 8
_NCHUNK = _RPW // _RPC
_CE = _RPC * _CW

_NP = 10240
_RPT = _NP // _NS
_CP = _NP // _NS

_mesh = plsc.VectorSubcoreMesh(core_axis_name="c", subcore_axis_name="s")


def _deg_body(dst_hbm, out_hbm, dst_v, hist_v, tmp_v, acc_v, slots_sh):
    c = lax.axis_index("c")
    s = lax.axis_index("s")
    w = c * _NS + s

    def zero_body(i, carry):
        hist_v[pl.ds(i * _L, _L)] = jnp.zeros((_L,), jnp.float32)
        return carry

    lax.fori_loop(0, _NP // _L, zero_body, 0)

    pltpu.sync_copy(dst_hbm.at[pl.ds(w * _RPW, _RPW)], dst_v)

    ones = jnp.ones((_L,), jnp.float32)

    def hist_body(r, carry):
        for k in range(_CW // _L):
            idx = dst_v[r, pl.ds(k * _L, _L)]
            plsc.addupdate_scatter(hist_v, [idx], ones)
        return carry

    lax.fori_loop(0, _RPW, hist_body, 0)

    pltpu.sync_copy(hist_v, slots_sh.at[s])
    plsc.subcore_barrier()

    for k in range(_CP // _L):
        acc_v[pl.ds(k * _L, _L)] = jnp.zeros((_L,), jnp.float32)
    for t in range(_NS):
        pltpu.sync_copy(slots_sh.at[t, pl.ds(s * _CP, _CP)], tmp_v)
        for k in range(_CP // _L):
            sl = pl.ds(k * _L, _L)
            acc_v[sl] = acc_v[sl] + tmp_v[sl]

    pltpu.sync_copy(acc_v, out_hbm.at[pl.ds(c * _NP + s * _CP, _CP)])


_deg_call = pl.kernel(
    _deg_body,
    out_type=jax.ShapeDtypeStruct((_NC * _NP,), jnp.float32),
    mesh=_mesh,
    compiler_params=pltpu.CompilerParams(needs_layout_passes=False),
    scratch_types=[
        pltpu.VMEM((_RPW, _CW), jnp.int32),
        pltpu.VMEM((_NP,), jnp.float32),
        pltpu.VMEM((_CP,), jnp.float32),
        pltpu.VMEM((_CP,), jnp.float32),
        pltpu.VMEM_SHARED((_NS, _NP), jnp.float32),
    ],
)


_SR = 4
_SCE = _SR * _CW
_NSUB = _RPW // _SR
_NPAIR = _NSUB // 2


def _agg_body(g_hbm, src_hbm, dst_hbm, out_hbm, idxs_v, idxd_v, rows0_v,
              rows1_v, acc_sh, *scratch, D, stage):
    if stage:
        g_sh, g0, g1, s0, s1 = scratch
    else:
        g0, g1, s0, s1 = scratch
        g_sh = None
    c = lax.axis_index("c")
    s = lax.axis_index("s")
    w = c * _NS + s

    if stage:
        pltpu.sync_copy(g_hbm.at[pl.ds(s * _RPT, _RPT)],
                        g_sh.at[pl.ds(s * _RPT, _RPT)])
    g_src = g_sh if stage else g_hbm

    def zero_body(r, carry):
        for k in range(D // _L):
            rows0_v[r, pl.ds(k * _L, _L)] = jnp.zeros((_L,), jnp.float32)
        return carry

    lax.fori_loop(0, _SCE, zero_body, 0)
    pltpu.sync_copy(rows0_v, acc_sh.at[pl.ds(s * _RPT, _SCE)])
    pltpu.sync_copy(rows0_v.at[pl.ds(0, _RPT - _SCE)],
                    acc_sh.at[pl.ds(s * _RPT + _SCE, _RPT - _SCE)])
    plsc.subcore_barrier()

    pltpu.sync_copy(src_hbm.at[pl.ds(w * _RPW, _RPW)], idxs_v)
    pltpu.sync_copy(dst_hbm.at[pl.ds(w * _RPW, _RPW)], idxd_v)

    def fire_gather(rbase, rows_v, sem):
        for k in range(_SR):
            pltpu.async_copy(g_src.at[idxs_v.at[rbase + k]],
                             rows_v.at[pl.ds(k * _CW, _CW)], sem)

    def fire_scatter(rbase, rows_v, sem):
        for k in range(_SR):
            pltpu.async_copy(rows_v.at[pl.ds(k * _CW, _CW)],
                             acc_sh.at[idxd_v.at[rbase + k]], sem, add=True)

    def drain(sem, rows_v):
        pltpu.make_async_copy(g_hbm.at[pl.ds(0, _SCE)], rows_v, sem).wait()

    fire_gather(0, rows0_v, g0)

    def pair_body(i, carry):
        base = 2 * i * _SR
        @pl.when(i > 0)
        def _():
            drain(s1, rows1_v)
        fire_gather(base + _SR, rows1_v, g1)
        drain(g0, rows0_v)
        fire_scatter(base, rows0_v, s0)
        drain(s0, rows0_v)
        @pl.when(i < _NPAIR - 1)
        def _():
            fire_gather(base + 2 * _SR, rows0_v, g0)
        drain(g1, rows1_v)
        fire_scatter(base + _SR, rows1_v, s1)
        return carry

    lax.fori_loop(0, _NPAIR, pair_body, 0)
    drain(s1, rows1_v)
    plsc.subcore_barrier()

    pltpu.sync_copy(acc_sh.at[pl.ds(s * _RPT, _RPT)],
                    out_hbm.at[pl.ds(c * _NP + s * _RPT, _RPT)])


def _make_agg(D, stage):
    scratch = [
        pltpu.VMEM((_RPW, _CW), jnp.int32),
        pltpu.VMEM((_RPW, _CW), jnp.int32),
        pltpu.VMEM((_SCE, D), jnp.float32),
        pltpu.VMEM((_SCE, D), jnp.float32),
        pltpu.VMEM_SHARED((_NP, D), jnp.float32),
    ]
    if stage:
        scratch.append(pltpu.VMEM_SHARED((_NP, D), jnp.float32))
    scratch += [pltpu.SemaphoreType.DMA] * 4
    return pl.kernel(
        functools.partial(_agg_body, D=D, stage=stage),
        out_type=jax.ShapeDtypeStruct((_NC * _NP, D), jnp.float32),
        mesh=_mesh,
        compiler_params=pltpu.CompilerParams(use_tc_tiling_on_sc=False),
        scratch_types=scratch,
    )


_agg64 = _make_agg(64, stage=False)
_agg32 = _make_agg(32, stage=True)


_RB = 1000
_GRID = (_N // _RB,)


def _k1_body(x_ref, w_ref, d0_ref, d1_ref, g_ref, dinv_ref):
    deg = d0_ref[...] + d1_ref[...] + 1.0
    dv = lax.rsqrt(deg)
    h = jnp.dot(x_ref[...], w_ref[...], preferred_element_type=jnp.float32)
    g_ref[...] = h * dv
    dinv_ref[...] = dv


def _k1(x, W1, d0, d1):
    F = x.shape[1]
    Do = W1.shape[1]
    return pl.pallas_call(
        _k1_body,
        grid=_GRID,
        in_specs=[
            pl.BlockSpec((_RB, F), lambda i: (i, 0)),
            pl.BlockSpec((F, Do), lambda i: (0, 0)),
            pl.BlockSpec((_RB, 1), lambda i: (i, 0)),
            pl.BlockSpec((_RB, 1), lambda i: (i, 0)),
        ],
        out_specs=[
            pl.BlockSpec((_RB, Do), lambda i: (i, 0)),
            pl.BlockSpec((_RB, 1), lambda i: (i, 0)),
        ],
        out_shape=[
            jax.ShapeDtypeStruct((_N, Do), jnp.float32),
            jax.ShapeDtypeStruct((_N, 1), jnp.float32),
        ],
    )(x, W1, d0, d1)


def _k2_body(p0_ref, p1_ref, g_ref, dinv_ref, b_ref, w_ref, out_ref):
    sacc = p0_ref[...] + p1_ref[...] + g_ref[...]
    dv = dinv_ref[...]
    h = jnp.maximum(sacc * dv + b_ref[...], 0.0)
    out_ref[...] = jnp.dot(h, w_ref[...],
                           preferred_element_type=jnp.float32) * dv


def _k2(p0, p1, g1, dinv, b1, W2):
    Di = p0.shape[1]
    Do = W2.shape[1]
    return pl.pallas_call(
        _k2_body,
        grid=_GRID,
        in_specs=[
            pl.BlockSpec((_RB, Di), lambda i: (i, 0)),
            pl.BlockSpec((_RB, Di), lambda i: (i, 0)),
            pl.BlockSpec((_RB, Di), lambda i: (i, 0)),
            pl.BlockSpec((_RB, 1), lambda i: (i, 0)),
            pl.BlockSpec((1, Di), lambda i: (0, 0)),
            pl.BlockSpec((Di, Do), lambda i: (0, 0)),
        ],
        out_specs=pl.BlockSpec((_RB, Do), lambda i: (i, 0)),
        out_shape=jax.ShapeDtypeStruct((_N, Do), jnp.float32),
    )(p0, p1, g1, dinv, b1, W2)


def _k3_body(q0_ref, q1_ref, g_ref, dinv_ref, b_ref, w_ref, bc_ref, out_ref):
    sacc = q0_ref[...] + q1_ref[...] + g_ref[...]
    h = jnp.maximum(sacc * dinv_ref[...] + b_ref[...], 0.0)
    out_ref[...] = jnp.dot(h, w_ref[...],
                           preferred_element_type=jnp.float32) + bc_ref[...]


def _k3(q0, q1, g2, dinv, b2, Wc, bc):
    Di = q0.shape[1]
    Do = Wc.shape[1]
    return pl.pallas_call(
        _k3_body,
        grid=_GRID,
        in_specs=[
            pl.BlockSpec((_RB, Di), lambda i: (i, 0)),
            pl.BlockSpec((_RB, Di), lambda i: (i, 0)),
            pl.BlockSpec((_RB, Di), lambda i: (i, 0)),
            pl.BlockSpec((_RB, 1), lambda i: (i, 0)),
            pl.BlockSpec((1, Di), lambda i: (0, 0)),
            pl.BlockSpec((Di, Do), lambda i: (0, 0)),
            pl.BlockSpec((1, Do), lambda i: (0, 0)),
        ],
        out_specs=pl.BlockSpec((_RB, Do), lambda i: (i, 0)),
        out_shape=jax.ShapeDtypeStruct((_N, Do), jnp.float32),
    )(q0, q1, g2, dinv, b2, Wc, bc)


@jax.jit
def kernel(x, edge_index, W1, b1, W2, b2, Wc, bc):
    src = jnp.pad(edge_index[0].reshape(_NW, _EPW), ((0, 0), (0, _PAD)),
                  constant_values=0).reshape(_ER, _CW)
    dst = jnp.pad(edge_index[1].reshape(_NW, _EPW), ((0, 0), (0, _PAD)),
                  constant_values=_NP - 1).reshape(_ER, _CW)

    degp = _deg_call(dst)
    d0 = degp[:_N, None]
    d1 = degp[_NP:_NP + _N, None]

    g1, dinv = _k1(x, W1, d0, d1)
    p = _agg64(g1, src, dst)
    g2 = _k2(p[:_N], p[_NP:_NP + _N], g1, dinv, b1.reshape(1, -1), W2)
    q = _agg32(g2, src, dst)
    return _k3(q[:_N], q[_NP:_NP + _N], g2, dinv, b2.reshape(1, -1), Wc,
               bc.reshape(1, -1))


# speedup vs baseline: 33.3901x; 1.3225x over previous
import functools

import jax
import jax.numpy as jnp
from jax import lax
from jax.experimental import pallas as pl
from jax.experimental.pallas import tpu as pltpu
from jax.experimental.pallas import tpu_sc as plsc

_NC, _NS, _L = 2, 16, 16
_NW = _NC * _NS

_N = 10000
_E = 320000

_CW = 128
_EPW = _E // _NW
_EPWP = 10240
_PAD = _EPWP - _EPW
_RPW = _EPWP // _CW
_ER = _NW * _RPW
_RPC = 8
_NCHUNK = _RPW // _RPC
_CE = _RPC * _CW

_NP = 10240
_RPT = _NP // _NS
_CP = _NP // _NS

_mesh = plsc.VectorSubcoreMesh(core_axis_name="c", subcore_axis_name="s")


def _deg_body(dst_hbm, out_hbm, dst_v, hist_v, tmp_v, acc_v, slots_sh):
    c = lax.axis_index("c")
    s = lax.axis_index("s")
    w = c * _NS + s

    def zero_body(i, carry):
        hist_v[pl.ds(i * _L, _L)] = jnp.zeros((_L,), jnp.float32)
        return carry

    lax.fori_loop(0, _NP // _L, zero_body, 0)

    pltpu.sync_copy(dst_hbm.at[pl.ds(w * _RPW, _RPW)], dst_v)

    ones = jnp.ones((_L,), jnp.float32)

    def hist_body(r, carry):
        for k in range(_CW // _L):
            idx = dst_v[r, pl.ds(k * _L, _L)]
            plsc.addupdate_scatter(hist_v, [idx], ones)
        return carry

    lax.fori_loop(0, _RPW, hist_body, 0)

    pltpu.sync_copy(hist_v, slots_sh.at[s])
    plsc.subcore_barrier()

    for k in range(_CP // _L):
        acc_v[pl.ds(k * _L, _L)] = jnp.zeros((_L,), jnp.float32)
    for t in range(_NS):
        pltpu.sync_copy(slots_sh.at[t, pl.ds(s * _CP, _CP)], tmp_v)
        for k in range(_CP // _L):
            sl = pl.ds(k * _L, _L)
            acc_v[sl] = acc_v[sl] + tmp_v[sl]

    pltpu.sync_copy(acc_v, out_hbm.at[pl.ds(c * _NP + s * _CP, _CP)])


_deg_call = pl.kernel(
    _deg_body,
    out_type=jax.ShapeDtypeStruct((_NC * _NP,), jnp.float32),
    mesh=_mesh,
    compiler_params=pltpu.CompilerParams(needs_layout_passes=False),
    scratch_types=[
        pltpu.VMEM((_RPW, _CW), jnp.int32),
        pltpu.VMEM((_NP,), jnp.float32),
        pltpu.VMEM((_CP,), jnp.float32),
        pltpu.VMEM((_CP,), jnp.float32),
        pltpu.VMEM_SHARED((_NS, _NP), jnp.float32),
    ],
)


_SR = 4
_SCE = _SR * _CW
_NSUB = _RPW // _SR
_NPAIR = _NSUB // 2


_D = 32


def _agg_body(*refs, nph):
    gs = refs[:nph]
    src_hbm, dst_hbm = refs[nph], refs[nph + 1]
    outs = refs[nph + 2:2 * nph + 2]
    (idxs_v, idxd_v, rows0_v, rows1_v, acc_sh, g_sh,
     g0, g1, s0, s1) = refs[2 * nph + 2:]
    c = lax.axis_index("c")
    s = lax.axis_index("s")
    w = c * _NS + s

    pltpu.sync_copy(src_hbm.at[pl.ds(w * _RPW, _RPW)], idxs_v)
    pltpu.sync_copy(dst_hbm.at[pl.ds(w * _RPW, _RPW)], idxd_v)

    def fire_gather(rbase, rows_v, sem):
        for k in range(_SR):
            pltpu.async_copy(g_sh.at[idxs_v.at[rbase + k]],
                             rows_v.at[pl.ds(k * _CW, _CW)], sem)

    def fire_scatter(rbase, rows_v, sem):
        for k in range(_SR):
            pltpu.async_copy(rows_v.at[pl.ds(k * _CW, _CW)],
                             acc_sh.at[idxd_v.at[rbase + k]], sem, add=True)

    def drain(sem, rows_v):
        pltpu.make_async_copy(gs[0].at[pl.ds(0, _SCE)], rows_v, sem).wait()

    for g_hbm_p, out_hbm_p in zip(gs, outs):
        pltpu.sync_copy(g_hbm_p.at[pl.ds(s * _RPT, _RPT)],
                        g_sh.at[pl.ds(s * _RPT, _RPT)])

        def zero_body(r, carry):
            for k in range(_D // _L):
                rows0_v[r, pl.ds(k * _L, _L)] = jnp.zeros((_L,), jnp.float32)
            return carry

        lax.fori_loop(0, _SCE, zero_body, 0)
        pltpu.sync_copy(rows0_v, acc_sh.at[pl.ds(s * _RPT, _SCE)])
        pltpu.sync_copy(rows0_v.at[pl.ds(0, _RPT - _SCE)],
                        acc_sh.at[pl.ds(s * _RPT + _SCE, _RPT - _SCE)])
        plsc.subcore_barrier()

        fire_gather(0, rows0_v, g0)

        def pair_body(i, carry):
            base = 2 * i * _SR
            @pl.when(i > 0)
            def _():
                drain(s1, rows1_v)
            fire_gather(base + _SR, rows1_v, g1)
            drain(g0, rows0_v)
            fire_scatter(base, rows0_v, s0)
            drain(s0, rows0_v)
            @pl.when(i < _NPAIR - 1)
            def _():
                fire_gather(base + 2 * _SR, rows0_v, g0)
            drain(g1, rows1_v)
            fire_scatter(base + _SR, rows1_v, s1)
            return carry

        lax.fori_loop(0, _NPAIR, pair_body, 0)
        drain(s1, rows1_v)
        plsc.subcore_barrier()

        pltpu.sync_copy(acc_sh.at[pl.ds(s * _RPT, _RPT)],
                        out_hbm_p.at[pl.ds(c * _NP + s * _RPT, _RPT)])


def _make_agg(nph):
    out = [jax.ShapeDtypeStruct((_NC * _NP, _D), jnp.float32)] * nph
    return pl.kernel(
        functools.partial(_agg_body, nph=nph),
        out_type=out if nph > 1 else out[0],
        mesh=_mesh,
        compiler_params=pltpu.CompilerParams(use_tc_tiling_on_sc=False),
        scratch_types=[
            pltpu.VMEM((_RPW, _CW), jnp.int32),
            pltpu.VMEM((_RPW, _CW), jnp.int32),
            pltpu.VMEM((_SCE, _D), jnp.float32),
            pltpu.VMEM((_SCE, _D), jnp.float32),
            pltpu.VMEM_SHARED((_NP, _D), jnp.float32),
            pltpu.VMEM_SHARED((_NP, _D), jnp.float32),
            pltpu.SemaphoreType.DMA,
            pltpu.SemaphoreType.DMA,
            pltpu.SemaphoreType.DMA,
            pltpu.SemaphoreType.DMA,
        ],
    )


_agg2 = _make_agg(2)
_agg1 = _make_agg(1)


_RB = 1000
_GRID = (_N // _RB,)


def _k1_body(x_ref, w_ref, d0_ref, d1_ref, glo_ref, ghi_ref, dinv_ref):
    deg = d0_ref[...] + d1_ref[...] + 1.0
    dv = lax.rsqrt(deg)
    h = jnp.dot(x_ref[...], w_ref[...], preferred_element_type=jnp.float32)
    g = h * dv
    glo_ref[...] = g[:, :_D]
    ghi_ref[...] = g[:, _D:]
    dinv_ref[...] = dv


def _k1(x, W1, d0, d1):
    F = x.shape[1]
    return pl.pallas_call(
        _k1_body,
        grid=_GRID,
        in_specs=[
            pl.BlockSpec((_RB, F), lambda i: (i, 0)),
            pl.BlockSpec((F, 2 * _D), lambda i: (0, 0)),
            pl.BlockSpec((_RB, 1), lambda i: (i, 0)),
            pl.BlockSpec((_RB, 1), lambda i: (i, 0)),
        ],
        out_specs=[
            pl.BlockSpec((_RB, _D), lambda i: (i, 0)),
            pl.BlockSpec((_RB, _D), lambda i: (i, 0)),
            pl.BlockSpec((_RB, 1), lambda i: (i, 0)),
        ],
        out_shape=[
            jax.ShapeDtypeStruct((_NP, _D), jnp.float32),
            jax.ShapeDtypeStruct((_NP, _D), jnp.float32),
            jax.ShapeDtypeStruct((_N, 1), jnp.float32),
        ],
    )(x, W1, d0, d1)


def _k2_body(pl0_ref, pl1_ref, ph0_ref, ph1_ref, gl_ref, gh_ref, dinv_ref,
             b_ref, w_ref, out_ref):
    slo = pl0_ref[...] + pl1_ref[...] + gl_ref[...]
    shi = ph0_ref[...] + ph1_ref[...] + gh_ref[...]
    sacc = jnp.concatenate([slo, shi], axis=1)
    dv = dinv_ref[...]
    h = jnp.maximum(sacc * dv + b_ref[...], 0.0)
    out_ref[...] = jnp.dot(h, w_ref[...],
                           preferred_element_type=jnp.float32) * dv


def _k2(pl0, pl1, ph0, ph1, gl, gh, dinv, b1, W2):
    Do = W2.shape[1]
    return pl.pallas_call(
        _k2_body,
        grid=_GRID,
        in_specs=[
            pl.BlockSpec((_RB, _D), lambda i: (i, 0)),
            pl.BlockSpec((_RB, _D), lambda i: (i, 0)),
            pl.BlockSpec((_RB, _D), lambda i: (i, 0)),
            pl.BlockSpec((_RB, _D), lambda i: (i, 0)),
            pl.BlockSpec((_RB, _D), lambda i: (i, 0)),
            pl.BlockSpec((_RB, _D), lambda i: (i, 0)),
            pl.BlockSpec((_RB, 1), lambda i: (i, 0)),
            pl.BlockSpec((1, 2 * _D), lambda i: (0, 0)),
            pl.BlockSpec((2 * _D, Do), lambda i: (0, 0)),
        ],
        out_specs=pl.BlockSpec((_RB, Do), lambda i: (i, 0)),
        out_shape=jax.ShapeDtypeStruct((_NP, Do), jnp.float32),
    )(pl0, pl1, ph0, ph1, gl, gh, dinv, b1, W2)


def _k3_body(q0_ref, q1_ref, g_ref, dinv_ref, b_ref, w_ref, bc_ref, out_ref):
    sacc = q0_ref[...] + q1_ref[...] + g_ref[...]
    h = jnp.maximum(sacc * dinv_ref[...] + b_ref[...], 0.0)
    out_ref[...] = jnp.dot(h, w_ref[...],
                           preferred_element_type=jnp.float32) + bc_ref[...]


def _k3(q0, q1, g2, dinv, b2, Wc, bc):
    Di = q0.shape[1]
    Do = Wc.shape[1]
    return pl.pallas_call(
        _k3_body,
        grid=_GRID,
        in_specs=[
            pl.BlockSpec((_RB, Di), lambda i: (i, 0)),
            pl.BlockSpec((_RB, Di), lambda i: (i, 0)),
            pl.BlockSpec((_RB, Di), lambda i: (i, 0)),
            pl.BlockSpec((_RB, 1), lambda i: (i, 0)),
            pl.BlockSpec((1, Di), lambda i: (0, 0)),
            pl.BlockSpec((Di, Do), lambda i: (0, 0)),
            pl.BlockSpec((1, Do), lambda i: (0, 0)),
        ],
        out_specs=pl.BlockSpec((_RB, Do), lambda i: (i, 0)),
        out_shape=jax.ShapeDtypeStruct((_N, Do), jnp.float32),
    )(q0, q1, g2, dinv, b2, Wc, bc)


@jax.jit
def kernel(x, edge_index, W1, b1, W2, b2, Wc, bc):
    src = jnp.pad(edge_index[0].reshape(_NW, _EPW), ((0, 0), (0, _PAD)),
                  constant_values=0).reshape(_ER, _CW)
    dst = jnp.pad(edge_index[1].reshape(_NW, _EPW), ((0, 0), (0, _PAD)),
                  constant_values=_NP - 1).reshape(_ER, _CW)

    degp = _deg_call(dst)
    d0 = degp[:_N, None]
    d1 = degp[_NP:_NP + _N, None]

    gl, gh, dinv = _k1(x, W1, d0, d1)
    p_lo, p_hi = _agg2(gl, gh, src, dst)
    g2 = _k2(p_lo[:_N], p_lo[_NP:_NP + _N], p_hi[:_N], p_hi[_NP:_NP + _N],
             gl[:_N], gh[:_N], dinv, b1.reshape(1, -1), W2)
    q = _agg1(g2, src, dst)
    return _k3(q[:_N], q[_NP:_NP + _N], g2[:_N], dinv, b2.reshape(1, -1), Wc,
               bc.reshape(1, -1))


# speedup vs baseline: 36.9255x; 1.1059x over previous
import functools

import jax
import jax.numpy as jnp
from jax import lax
from jax.experimental import pallas as pl
from jax.experimental.pallas import tpu as pltpu
from jax.experimental.pallas import tpu_sc as plsc

_NC, _NS, _L = 2, 16, 16
_NW = _NC * _NS

_N = 10000
_E = 320000

_CW = 128
_EPW = _E // _NW
_EPWP = 10240
_PAD = _EPWP - _EPW
_RPW = _EPWP // _CW
_ER = _NW * _RPW
_RPC = 8
_NCHUNK = _RPW // _RPC
_CE = _RPC * _CW

_NP = 10240
_RPT = _NP // _NS
_CP = _NP // _NS

_mesh = plsc.VectorSubcoreMesh(core_axis_name="c", subcore_axis_name="s")


def _deg_body(dst_hbm, out_hbm, dst_v, hist_v, tmp_v, acc_v, slots_sh):
    c = lax.axis_index("c")
    s = lax.axis_index("s")
    w = c * _NS + s

    def zero_body(i, carry):
        hist_v[pl.ds(i * _L, _L)] = jnp.zeros((_L,), jnp.float32)
        return carry

    lax.fori_loop(0, _NP // _L, zero_body, 0)

    pltpu.sync_copy(dst_hbm.at[pl.ds(w * _RPW, _RPW)], dst_v)

    ones = jnp.ones((_L,), jnp.float32)

    def hist_body(r, carry):
        for k in range(_CW // _L):
            idx = dst_v[r, pl.ds(k * _L, _L)]
            plsc.addupdate_scatter(hist_v, [idx], ones)
        return carry

    lax.fori_loop(0, _RPW, hist_body, 0)

    pltpu.sync_copy(hist_v, slots_sh.at[s])
    plsc.subcore_barrier()

    for k in range(_CP // _L):
        acc_v[pl.ds(k * _L, _L)] = jnp.zeros((_L,), jnp.float32)
    for t in range(_NS):
        pltpu.sync_copy(slots_sh.at[t, pl.ds(s * _CP, _CP)], tmp_v)
        for k in range(_CP // _L):
            sl = pl.ds(k * _L, _L)
            acc_v[sl] = acc_v[sl] + tmp_v[sl]

    pltpu.sync_copy(acc_v, out_hbm.at[pl.ds(c * _NP + s * _CP, _CP)])


_deg_call = pl.kernel(
    _deg_body,
    out_type=jax.ShapeDtypeStruct((_NC * _NP,), jnp.float32),
    mesh=_mesh,
    compiler_params=pltpu.CompilerParams(needs_layout_passes=False),
    scratch_types=[
        pltpu.VMEM((_RPW, _CW), jnp.int32),
        pltpu.VMEM((_NP,), jnp.float32),
        pltpu.VMEM((_CP,), jnp.float32),
        pltpu.VMEM((_CP,), jnp.float32),
        pltpu.VMEM_SHARED((_NS, _NP), jnp.float32),
    ],
)


_SR = 4
_SCE = _SR * _CW
_NSUB = _RPW // _SR
_NPAIR = _NSUB // 2


_D = 32


def _agg_body(*refs, nph):
    gs = refs[:nph]
    src_hbm, dst_hbm = refs[nph], refs[nph + 1]
    outs = refs[nph + 2:2 * nph + 2]
    (idxs_v, idxd_v, rows0_v, rows1_v, acc_sh, g_sh,
     g0, g1, s0, s1) = refs[2 * nph + 2:]
    c = lax.axis_index("c")
    s = lax.axis_index("s")
    w = c * _NS + s

    pltpu.sync_copy(src_hbm.at[pl.ds(w * _RPW, _RPW)], idxs_v)
    pltpu.sync_copy(dst_hbm.at[pl.ds(w * _RPW, _RPW)], idxd_v)

    def fire_gather(rbase, rows_v, sem):
        for k in range(_SR):
            pltpu.async_copy(g_sh.at[idxs_v.at[rbase + k]],
                             rows_v.at[pl.ds(k * _CW, _CW)], sem)

    def fire_scatter(rbase, rows_v, sem):
        for k in range(_SR):
            pltpu.async_copy(rows_v.at[pl.ds(k * _CW, _CW)],
                             acc_sh.at[idxd_v.at[rbase + k]], sem, add=True)

    def drain(sem, rows_v):
        pltpu.make_async_copy(gs[0].at[pl.ds(0, _SCE)], rows_v, sem).wait()

    for g_hbm_p, out_hbm_p in zip(gs, outs):
        pltpu.sync_copy(g_hbm_p.at[pl.ds(s * _RPT, _RPT)],
                        g_sh.at[pl.ds(s * _RPT, _RPT)])

        def zero_body(r, carry):
            for k in range(_D // _L):
                rows0_v[r, pl.ds(k * _L, _L)] = jnp.zeros((_L,), jnp.float32)
            return carry

        lax.fori_loop(0, _SCE, zero_body, 0)
        pltpu.sync_copy(rows0_v, acc_sh.at[pl.ds(s * _RPT, _SCE)])
        pltpu.sync_copy(rows0_v.at[pl.ds(0, _RPT - _SCE)],
                        acc_sh.at[pl.ds(s * _RPT + _SCE, _RPT - _SCE)])
        plsc.subcore_barrier()

        fire_gather(0, rows0_v, g0)

        def pair_body(i, carry):
            base = 2 * i * _SR
            @pl.when(i > 0)
            def _():
                drain(s1, rows1_v)
            fire_gather(base + _SR, rows1_v, g1)
            drain(g0, rows0_v)
            fire_scatter(base, rows0_v, s0)
            drain(s0, rows0_v)
            @pl.when(i < _NPAIR - 1)
            def _():
                fire_gather(base + 2 * _SR, rows0_v, g0)
            drain(g1, rows1_v)
            fire_scatter(base + _SR, rows1_v, s1)
            return carry

        lax.fori_loop(0, _NPAIR, pair_body, 0)
        drain(s1, rows1_v)
        plsc.subcore_barrier()

        pltpu.sync_copy(acc_sh.at[pl.ds(s * _RPT, _RPT)],
                        out_hbm_p.at[pl.ds(c * _NP + s * _RPT, _RPT)])


def _make_agg(nph):
    out = [jax.ShapeDtypeStruct((_NC * _NP, _D), jnp.float32)] * nph
    return pl.kernel(
        functools.partial(_agg_body, nph=nph),
        out_type=out if nph > 1 else out[0],
        mesh=_mesh,
        compiler_params=pltpu.CompilerParams(use_tc_tiling_on_sc=False),
        scratch_types=[
            pltpu.VMEM((_RPW, _CW), jnp.int32),
            pltpu.VMEM((_RPW, _CW), jnp.int32),
            pltpu.VMEM((_SCE, _D), jnp.float32),
            pltpu.VMEM((_SCE, _D), jnp.float32),
            pltpu.VMEM_SHARED((_NP, _D), jnp.float32),
            pltpu.VMEM_SHARED((_NP, _D), jnp.float32),
            pltpu.SemaphoreType.DMA,
            pltpu.SemaphoreType.DMA,
            pltpu.SemaphoreType.DMA,
            pltpu.SemaphoreType.DMA,
        ],
    )


_agg2 = _make_agg(2)
_agg1 = _make_agg(1)


_RB = 1024
_GRID = (_NP // _RB,)


_HB = _NP // _RB


def _k1_body(x_ref, w_ref, d0_ref, d1_ref, glo_ref, ghi_ref, dinv_ref):
    deg = d0_ref[...] + d1_ref[...] + 1.0
    dv = lax.rsqrt(deg)
    h = jnp.dot(x_ref[...], w_ref[...], preferred_element_type=jnp.float32)
    g = h * dv
    glo_ref[...] = g[:, :_D]
    ghi_ref[...] = g[:, _D:]
    dinv_ref[...] = dv


def _k1(x, W1, degp):
    F = x.shape[1]
    return pl.pallas_call(
        _k1_body,
        grid=_GRID,
        in_specs=[
            pl.BlockSpec((_RB, F), lambda i: (i, 0)),
            pl.BlockSpec((F, 2 * _D), lambda i: (0, 0)),
            pl.BlockSpec((_RB, 1), lambda i: (i, 0)),
            pl.BlockSpec((_RB, 1), lambda i: (i + _HB, 0)),
        ],
        out_specs=[
            pl.BlockSpec((_RB, _D), lambda i: (i, 0)),
            pl.BlockSpec((_RB, _D), lambda i: (i, 0)),
            pl.BlockSpec((_RB, 1), lambda i: (i, 0)),
        ],
        out_shape=[
            jax.ShapeDtypeStruct((_NP, _D), jnp.float32),
            jax.ShapeDtypeStruct((_NP, _D), jnp.float32),
            jax.ShapeDtypeStruct((_NP, 1), jnp.float32),
        ],
    )(x, W1, degp, degp)


def _k2_body(pl0_ref, pl1_ref, ph0_ref, ph1_ref, gl_ref, gh_ref, dinv_ref,
             b_ref, w_ref, out_ref):
    slo = pl0_ref[...] + pl1_ref[...] + gl_ref[...]
    shi = ph0_ref[...] + ph1_ref[...] + gh_ref[...]
    sacc = jnp.concatenate([slo, shi], axis=1)
    dv = dinv_ref[...]
    h = jnp.maximum(sacc * dv + b_ref[...], 0.0)
    out_ref[...] = jnp.dot(h, w_ref[...],
                           preferred_element_type=jnp.float32) * dv


def _k2(p_lo, p_hi, gl, gh, dinv, b1, W2):
    Do = W2.shape[1]
    return pl.pallas_call(
        _k2_body,
        grid=_GRID,
        in_specs=[
            pl.BlockSpec((_RB, _D), lambda i: (i, 0)),
            pl.BlockSpec((_RB, _D), lambda i: (i + _HB, 0)),
            pl.BlockSpec((_RB, _D), lambda i: (i, 0)),
            pl.BlockSpec((_RB, _D), lambda i: (i + _HB, 0)),
            pl.BlockSpec((_RB, _D), lambda i: (i, 0)),
            pl.BlockSpec((_RB, _D), lambda i: (i, 0)),
            pl.BlockSpec((_RB, 1), lambda i: (i, 0)),
            pl.BlockSpec((1, 2 * _D), lambda i: (0, 0)),
            pl.BlockSpec((2 * _D, Do), lambda i: (0, 0)),
        ],
        out_specs=pl.BlockSpec((_RB, Do), lambda i: (i, 0)),
        out_shape=jax.ShapeDtypeStruct((_NP, Do), jnp.float32),
    )(p_lo, p_lo, p_hi, p_hi, gl, gh, dinv, b1, W2)


def _k3_body(q0_ref, q1_ref, g_ref, dinv_ref, b_ref, w_ref, bc_ref, out_ref):
    sacc = q0_ref[...] + q1_ref[...] + g_ref[...]
    h = jnp.maximum(sacc * dinv_ref[...] + b_ref[...], 0.0)
    out_ref[...] = jnp.dot(h, w_ref[...],
                           preferred_element_type=jnp.float32) + bc_ref[...]


def _k3(q, g2, dinv, b2, Wc, bc):
    Do = Wc.shape[1]
    return pl.pallas_call(
        _k3_body,
        grid=_GRID,
        in_specs=[
            pl.BlockSpec((_RB, _D), lambda i: (i, 0)),
            pl.BlockSpec((_RB, _D), lambda i: (i + _HB, 0)),
            pl.BlockSpec((_RB, _D), lambda i: (i, 0)),
            pl.BlockSpec((_RB, 1), lambda i: (i, 0)),
            pl.BlockSpec((1, _D), lambda i: (0, 0)),
            pl.BlockSpec((_D, Do), lambda i: (0, 0)),
            pl.BlockSpec((1, Do), lambda i: (0, 0)),
        ],
        out_specs=pl.BlockSpec((_RB, Do), lambda i: (i, 0)),
        out_shape=jax.ShapeDtypeStruct((_N, Do), jnp.float32),
    )(q, q, g2, dinv, b2, Wc, bc)


@jax.jit
def kernel(x, edge_index, W1, b1, W2, b2, Wc, bc):
    src = jnp.pad(edge_index[0].reshape(_NW, _EPW), ((0, 0), (0, _PAD)),
                  constant_values=0).reshape(_ER, _CW)
    dst = jnp.pad(edge_index[1].reshape(_NW, _EPW), ((0, 0), (0, _PAD)),
                  constant_values=_NP - 1).reshape(_ER, _CW)

    degp = _deg_call(dst).reshape(_NC * _NP, 1)

    gl, gh, dinv = _k1(x, W1, degp)
    p_lo, p_hi = _agg2(gl, gh, src, dst)
    g2 = _k2(p_lo, p_hi, gl, gh, dinv, b1.reshape(1, -1), W2)
    q = _agg1(g2, src, dst)
    return _k3(q, g2, dinv, b2.reshape(1, -1), Wc, bc.reshape(1, -1))


# speedup vs baseline: 41.4665x; 1.1230x over previous
import functools

import jax
import jax.numpy as jnp
from jax import lax
from jax.experimental import pallas as pl
from jax.experimental.pallas import tpu as pltpu
from jax.experimental.pallas import tpu_sc as plsc

_NC, _NS, _L = 2, 16, 16
_NW = _NC * _NS

_N = 10000
_E = 320000

_CW = 128
_EPW = _E // _NW
_EPWP = 10240
_PAD = _EPWP - _EPW
_RPW = _EPWP // _CW
_ER = _NW * _RPW
_RPC = 8
_NCHUNK = _RPW // _RPC
_CE = _RPC * _CW

_NP = 10240
_RPT = _NP // _NS
_CP = _NP // _NS

_mesh = plsc.VectorSubcoreMesh(core_axis_name="c", subcore_axis_name="s")


def _deg_body(dst_hbm, out_hbm, dst_v, hist_v, tmp_v, acc_v, slots_sh):
    c = lax.axis_index("c")
    s = lax.axis_index("s")
    w = c * _NS + s

    def zero_body(i, carry):
        hist_v[pl.ds(i * _L, _L)] = jnp.zeros((_L,), jnp.float32)
        return carry

    lax.fori_loop(0, _NP // _L, zero_body, 0)

    pltpu.sync_copy(dst_hbm.at[pl.ds(w * _RPW, _RPW)], dst_v)

    ones = jnp.ones((_L,), jnp.float32)

    def hist_body(r, carry):
        for k in range(_CW // _L):
            idx = dst_v[r, pl.ds(k * _L, _L)]
            plsc.addupdate_scatter(hist_v, [idx], ones)
        return carry

    lax.fori_loop(0, _RPW, hist_body, 0)

    pltpu.sync_copy(hist_v, slots_sh.at[s])
    plsc.subcore_barrier()

    for k in range(_CP // _L):
        acc_v[pl.ds(k * _L, _L)] = jnp.zeros((_L,), jnp.float32)
    for t in range(_NS):
        pltpu.sync_copy(slots_sh.at[t, pl.ds(s * _CP, _CP)], tmp_v)
        for k in range(_CP // _L):
            sl = pl.ds(k * _L, _L)
            acc_v[sl] = acc_v[sl] + tmp_v[sl]

    pltpu.sync_copy(acc_v, out_hbm.at[pl.ds(c * _NP + s * _CP, _CP)])


_deg_call = pl.kernel(
    _deg_body,
    out_type=jax.ShapeDtypeStruct((_NC * _NP,), jnp.float32),
    mesh=_mesh,
    compiler_params=pltpu.CompilerParams(needs_layout_passes=False),
    scratch_types=[
        pltpu.VMEM((_RPW, _CW), jnp.int32),
        pltpu.VMEM((_NP,), jnp.float32),
        pltpu.VMEM((_CP,), jnp.float32),
        pltpu.VMEM((_CP,), jnp.float32),
        pltpu.VMEM_SHARED((_NS, _NP), jnp.float32),
    ],
)


_SR = 4
_SCE = _SR * _CW
_NSUB = _RPW // _SR
_NPAIR = _NSUB // 2


_D = 32


def _agg_body(*refs, nph):
    g_hbm, src_hbm, dst_hbm, out_hbm = refs[:4]
    (idxs_v, idxd_v, rows0_v, rows1_v, acc_sh, g_sh,
     g0, g1, s0, s1) = refs[4:]
    c = lax.axis_index("c")
    s = lax.axis_index("s")
    w = c * _NS + s

    pltpu.sync_copy(src_hbm.at[pl.ds(w * _RPW, _RPW)], idxs_v)
    pltpu.sync_copy(dst_hbm.at[pl.ds(w * _RPW, _RPW)], idxd_v)

    def fire_gather(rbase, rows_v, sem):
        for k in range(_SR):
            pltpu.async_copy(g_sh.at[idxs_v.at[rbase + k]],
                             rows_v.at[pl.ds(k * _CW, _CW)], sem)

    def fire_scatter(rbase, rows_v, sem):
        for k in range(_SR):
            pltpu.async_copy(rows_v.at[pl.ds(k * _CW, _CW)],
                             acc_sh.at[idxd_v.at[rbase + k]], sem, add=True)

    def drain(sem, rows_v):
        pltpu.make_async_copy(g_hbm.at[pl.ds(0, _SCE), pl.ds(0, _D)],
                              rows_v, sem).wait()

    for ph in range(nph):
        pltpu.sync_copy(g_hbm.at[pl.ds(s * _RPT, _RPT), pl.ds(ph * _D, _D)],
                        g_sh.at[pl.ds(s * _RPT, _RPT)])

        def zero_body(r, carry):
            for k in range(_D // _L):
                rows0_v[r, pl.ds(k * _L, _L)] = jnp.zeros((_L,), jnp.float32)
            return carry

        lax.fori_loop(0, _SCE, zero_body, 0)
        pltpu.sync_copy(rows0_v, acc_sh.at[pl.ds(s * _RPT, _SCE)])
        pltpu.sync_copy(rows0_v.at[pl.ds(0, _RPT - _SCE)],
                        acc_sh.at[pl.ds(s * _RPT + _SCE, _RPT - _SCE)])
        plsc.subcore_barrier()

        fire_gather(0, rows0_v, g0)

        def pair_body(i, carry):
            base = 2 * i * _SR
            @pl.when(i > 0)
            def _():
                drain(s1, rows1_v)
            fire_gather(base + _SR, rows1_v, g1)
            drain(g0, rows0_v)
            fire_scatter(base, rows0_v, s0)
            drain(s0, rows0_v)
            @pl.when(i < _NPAIR - 1)
            def _():
                fire_gather(base + 2 * _SR, rows0_v, g0)
            drain(g1, rows1_v)
            fire_scatter(base + _SR, rows1_v, s1)
            return carry

        lax.fori_loop(0, _NPAIR, pair_body, 0)
        drain(s1, rows1_v)
        plsc.subcore_barrier()

        pltpu.sync_copy(acc_sh.at[pl.ds(s * _RPT, _RPT)],
                        out_hbm.at[pl.ds(c * _NP + s * _RPT, _RPT),
                                   pl.ds(ph * _D, _D)])


def _make_agg(nph):
    return pl.kernel(
        functools.partial(_agg_body, nph=nph),
        out_type=jax.ShapeDtypeStruct((_NC * _NP, 4 * _D), jnp.float32),
        mesh=_mesh,
        compiler_params=pltpu.CompilerParams(use_tc_tiling_on_sc=False),
        scratch_types=[
            pltpu.VMEM((_RPW, _CW), jnp.int32),
            pltpu.VMEM((_RPW, _CW), jnp.int32),
            pltpu.VMEM((_SCE, _D), jnp.float32),
            pltpu.VMEM((_SCE, _D), jnp.float32),
            pltpu.VMEM_SHARED((_NP, _D), jnp.float32),
            pltpu.VMEM_SHARED((_NP, _D), jnp.float32),
            pltpu.SemaphoreType.DMA,
            pltpu.SemaphoreType.DMA,
            pltpu.SemaphoreType.DMA,
            pltpu.SemaphoreType.DMA,
        ],
    )


_agg2 = _make_agg(2)
_agg1 = _make_agg(1)


_RB = 1024
_GRID = (_NP // _RB,)


_HB = _NP // _RB


def _k1_body(x_ref, w_ref, d0_ref, d1_ref, g_ref, dinv_ref):
    deg = d0_ref[...] + d1_ref[...] + 1.0
    dv = lax.rsqrt(deg)
    h = jnp.dot(x_ref[...], w_ref[...], preferred_element_type=jnp.float32)
    g = h * dv
    g_ref[...] = jnp.concatenate(
        [g, jnp.broadcast_to(dv, (g.shape[0], 2 * _D))], axis=1)
    dinv_ref[...] = dv


def _k1(x, W1, degp):
    F = x.shape[1]
    return pl.pallas_call(
        _k1_body,
        grid=_GRID,
        in_specs=[
            pl.BlockSpec((_RB, F), lambda i: (i, 0)),
            pl.BlockSpec((F, 2 * _D), lambda i: (0, 0)),
            pl.BlockSpec((_RB, 1), lambda i: (i, 0)),
            pl.BlockSpec((_RB, 1), lambda i: (i + _HB, 0)),
        ],
        out_specs=[
            pl.BlockSpec((_RB, 4 * _D), lambda i: (i, 0)),
            pl.BlockSpec((_RB, 1), lambda i: (i, 0)),
        ],
        out_shape=[
            jax.ShapeDtypeStruct((_NP, 4 * _D), jnp.float32),
            jax.ShapeDtypeStruct((_NP, 1), jnp.float32),
        ],
    )(x, W1, degp, degp)


def _k2_body(p0_ref, p1_ref, g1_ref, dinv_ref, b_ref, w_ref, out_ref):
    sacc = (p0_ref[...][:, :2 * _D] + p1_ref[...][:, :2 * _D]
            + g1_ref[...][:, :2 * _D])
    dv = dinv_ref[...]
    h = jnp.maximum(sacc * dv + b_ref[...], 0.0)
    g2 = jnp.dot(h, w_ref[...], preferred_element_type=jnp.float32) * dv
    out_ref[...] = jnp.concatenate(
        [g2, jnp.broadcast_to(dv, (g2.shape[0], 3 * _D))], axis=1)


def _k2(p1, g1, dinv, b1, W2):
    return pl.pallas_call(
        _k2_body,
        grid=_GRID,
        in_specs=[
            pl.BlockSpec((_RB, 4 * _D), lambda i: (i, 0)),
            pl.BlockSpec((_RB, 4 * _D), lambda i: (i + _HB, 0)),
            pl.BlockSpec((_RB, 4 * _D), lambda i: (i, 0)),
            pl.BlockSpec((_RB, 1), lambda i: (i, 0)),
            pl.BlockSpec((1, 2 * _D), lambda i: (0, 0)),
            pl.BlockSpec((2 * _D, _D), lambda i: (0, 0)),
        ],
        out_specs=pl.BlockSpec((_RB, 4 * _D), lambda i: (i, 0)),
        out_shape=jax.ShapeDtypeStruct((_NP, 4 * _D), jnp.float32),
    )(p1, p1, g1, dinv, b1, W2)


def _k3_body(q0_ref, q1_ref, g_ref, dinv_ref, b_ref, w_ref, bc_ref, out_ref):
    sacc = (q0_ref[...][:, :_D] + q1_ref[...][:, :_D] + g_ref[...][:, :_D])
    h = jnp.maximum(sacc * dinv_ref[...] + b_ref[...], 0.0)
    out_ref[...] = jnp.dot(h, w_ref[...],
                           preferred_element_type=jnp.float32) + bc_ref[...]


def _k3(q, g2, dinv, b2, Wc, bc):
    Do = Wc.shape[1]
    return pl.pallas_call(
        _k3_body,
        grid=_GRID,
        in_specs=[
            pl.BlockSpec((_RB, 4 * _D), lambda i: (i, 0)),
            pl.BlockSpec((_RB, 4 * _D), lambda i: (i + _HB, 0)),
            pl.BlockSpec((_RB, 4 * _D), lambda i: (i, 0)),
            pl.BlockSpec((_RB, 1), lambda i: (i, 0)),
            pl.BlockSpec((1, _D), lambda i: (0, 0)),
            pl.BlockSpec((_D, Do), lambda i: (0, 0)),
            pl.BlockSpec((1, Do), lambda i: (0, 0)),
        ],
        out_specs=pl.BlockSpec((_RB, Do), lambda i: (i, 0)),
        out_shape=jax.ShapeDtypeStruct((_N, Do), jnp.float32),
    )(q, q, g2, dinv, b2, Wc, bc)


@jax.jit
def kernel(x, edge_index, W1, b1, W2, b2, Wc, bc):
    src = jnp.pad(edge_index[0].reshape(_NW, _EPW), ((0, 0), (0, _PAD)),
                  constant_values=0).reshape(_ER, _CW)
    dst = jnp.pad(edge_index[1].reshape(_NW, _EPW), ((0, 0), (0, _PAD)),
                  constant_values=_NP - 1).reshape(_ER, _CW)

    degp = _deg_call(dst).reshape(_NC * _NP, 1)

    g1, dinv = _k1(x, W1, degp)
    p1 = _agg2(g1, src, dst)
    g2 = _k2(p1, g1, dinv, b1.reshape(1, -1), W2)
    q = _agg1(g2, src, dst)
    return _k3(q, g2, dinv, b2.reshape(1, -1), Wc, bc.reshape(1, -1))
